# baseline (speedup 1.0000x reference)
import functools

import jax
import jax.numpy as jnp
from jax import lax
from jax.experimental import pallas as pl
from jax.experimental.pallas import tpu as pltpu
from jax.experimental.pallas import tpu_sc as plsc

N = 10000
E = 320000
D = 128
H = 512
G = 64

NC = 2
NS = 16
NW = NC * NS
EB = 64
NBUF = 4
STG = 2560
SB = STG // EB
NSTG0 = 8
NSTG1 = 0
EPW0 = NSTG0 * STG
EPW1 = NSTG1 * STG
DB = 128
NDB = 80
EPW = EPW0 + EPW1
E_PAD = NS * EPW
N_ACC = 10240
ACC_PER_TILE = N_ACC // NS
ZB = 32

_mesh = plsc.VectorSubcoreMesh(
    core_axis_name="c", subcore_axis_name="s", num_cores=NC, num_subcores=NS
)


def _wid(c, s):
    return s * NC + c


@functools.partial(
    pl.kernel,
    out_type=jax.ShapeDtypeStruct((NC, N_ACC, 128), jnp.float32),
    mesh=_mesh,
    scratch_types=[
        pltpu.VMEM((NDB, DB), jnp.int32),
        pltpu.VMEM((DB, 128), jnp.float32),
        pltpu.VMEM((ZB, 128), jnp.float32),
        pltpu.VMEM_SHARED((N_ACC, 128), jnp.float32),
    ],
)
def _sc_degree(dst3_hbm, ones_hbm, zeros_hbm, out_hbm, dstb, onesv, zb, acc):
    c = lax.axis_index("c")
    s = lax.axis_index("s")
    w = _wid(c, s)

    pltpu.sync_copy(dst3_hbm.at[w], dstb)
    pltpu.sync_copy(ones_hbm, onesv)
    pltpu.sync_copy(zeros_hbm, zb)
    for z in range(ACC_PER_TILE // ZB):
        pltpu.sync_copy(zb, acc.at[pl.ds(s * ACC_PER_TILE + z * ZB, ZB)])
    plsc.subcore_barrier()

    def body(b, carry):
        pltpu.sync_copy(onesv, acc.at[dstb.at[b]], add=True)
        return carry

    lax.fori_loop(0, NDB, body, 0)
    plsc.subcore_barrier()
    pltpu.sync_copy(
        acc.at[pl.ds(s * ACC_PER_TILE, ACC_PER_TILE)],
        out_hbm.at[c, pl.ds(s * ACC_PER_TILE, ACC_PER_TILE)],
    )


def _make_sc_agg(nch):
    @functools.partial(
        pl.kernel,
        out_type=jax.ShapeDtypeStruct((NC, nch, N_ACC, 128), jnp.float32),
        mesh=_mesh,
        scratch_types=[
            pltpu.VMEM((STG,), jnp.int32),
            pltpu.VMEM((SB, EB), jnp.int32),
            pltpu.VMEM((NBUF, EB, 128), jnp.float32),
            pltpu.VMEM((ZB, 128), jnp.float32),
            pltpu.VMEM_SHARED((N_ACC, 128), jnp.float32),
            [pltpu.SemaphoreType.DMA] * NBUF,
            [pltpu.SemaphoreType.DMA] * NBUF,
        ],
    )
    def _sc_agg(vflat_hbm, src_hbm, dst2_hbm, zeros_hbm, out_hbm,
                gidx, dstb, rows, zb, acc, semg, sems):
        c = lax.axis_index("c")
        s = lax.axis_index("s")
        base = c * (NS * EPW0) + s * jnp.where(c == 0, EPW0, EPW1)
        nst = jnp.where(c == 0, NSTG0, NSTG1)

        pltpu.sync_copy(zeros_hbm, zb)
        for z in range(ACC_PER_TILE // ZB):
            pltpu.sync_copy(zb, acc.at[pl.ds(s * ACC_PER_TILE + z * ZB, ZB)])
        plsc.subcore_barrier()

        def gather_start(b, j):
            pltpu.async_copy(
                vflat_hbm.at[gidx.at[pl.ds(b * EB, EB)]], rows.at[j], semg[j]
            )

        def gather_wait(b, j):
            pltpu.make_async_copy(
                vflat_hbm.at[gidx.at[pl.ds(b * EB, EB)]], rows.at[j], semg[j]
            ).wait()

        def scat_start(b, j):
            pltpu.async_copy(
                rows.at[j], acc.at[dstb.at[b]], sems[j], add=True
            )

        def scat_wait(b, j):
            pltpu.make_async_copy(
                rows.at[j], acc.at[dstb.at[b]], sems[j]
            ).wait()

        def do_stage(k, stg):
                ebase = pl.multiple_of(base + stg * STG, STG)
                rbase = pl.multiple_of((base + stg * STG) // EB, SB)
                pltpu.sync_copy(src_hbm.at[pl.ds(ebase, STG)], gidx)
                pltpu.sync_copy(dst2_hbm.at[pl.ds(rbase, SB)], dstb)
                if nch > 1:
                    def mkidx(jj, carry):
                        sl = pl.ds(jj * 16, 16)
                        gidx[sl] = gidx[sl] * nch + k
                        return carry

                    lax.fori_loop(0, STG // 16, mkidx, 0)

                gather_start(0, 0)
                gather_start(1, 1)

                def body(b2, carry):
                    for j in range(NBUF):
                        b = NBUF * b2 + j
                        jj = (j + 2) % NBUF
                        gather_wait(b, j)
                        scat_start(b, j)
                        if j >= 2:
                            scat_wait(b - 2, jj)
                        else:

                            @pl.when(b2 > 0)
                            def _():
                                scat_wait(b - 2, jj)

                        @pl.when(b + 2 < SB)
                        def _():
                            gather_start(b + 2, jj)

                    return carry

                lax.fori_loop(0, SB // NBUF, body, 0)
                scat_wait(SB - 2, (SB - 2) % NBUF)
                scat_wait(SB - 1, (SB - 1) % NBUF)

        for k in range(nch):
            for stg in range(max(NSTG0, NSTG1)):

                @pl.when(stg < nst)
                def _():
                    do_stage(k, stg)

            plsc.subcore_barrier()
            pltpu.sync_copy(
                acc.at[pl.ds(s * ACC_PER_TILE, ACC_PER_TILE)],
                out_hbm.at[c, k, pl.ds(s * ACC_PER_TILE, ACC_PER_TILE)],
            )
            if k + 1 < nch:
                for z in range(ACC_PER_TILE // ZB):
                    pltpu.sync_copy(
                        zb, acc.at[pl.ds(s * ACC_PER_TILE + z * ZB, ZB)]
                    )
            plsc.subcore_barrier()

    return _sc_agg


_sc_agg1 = _make_sc_agg(1)
_sc_agg4 = _make_sc_agg(4)


BN = 1000


def _prep_body(deg_ref, x_ref, dinv_ref, v1_ref):
    cnt = deg_ref[0, :, 0:1] + deg_ref[1, :, 0:1]
    dinv = lax.rsqrt(1.0 + cnt)
    dinv_ref[...] = dinv
    v1_ref[...] = x_ref[...] * dinv


def _tc_prep(deg, x):
    return pl.pallas_call(
        _prep_body,
        grid=(N // BN,),
        in_specs=[
            pl.BlockSpec((NC, BN, 128), lambda i: (0, i, 0)),
            pl.BlockSpec((BN, D), lambda i: (i, 0)),
        ],
        out_specs=[
            pl.BlockSpec((BN, 1), lambda i: (i, 0)),
            pl.BlockSpec((BN, D), lambda i: (i, 0)),
        ],
        out_shape=[
            jax.ShapeDtypeStruct((N, 1), jnp.float32),
            jax.ShapeDtypeStruct((N, D), jnp.float32),
        ],
    )(deg, x)


def _combine(p_ref, v_ref, dinv_ref):
    nch = p_ref.shape[1]
    parts = [p_ref[0, k] + p_ref[1, k] for k in range(nch)]
    agg = parts[0] if nch == 1 else jnp.concatenate(parts, axis=1)
    return (agg + v_ref[...]) * dinv_ref[...]


def _layer_body(p_ref, v_ref, dinv_ref, w_ref, b_ref, out_ref):
    z = _combine(p_ref, v_ref, dinv_ref)
    h = jnp.dot(z, w_ref[...], preferred_element_type=jnp.float32) + b_ref[...]
    out_ref[...] = jnp.maximum(h, 0.0) * dinv_ref[...]


def _tc_layer(p, v, dinv, w, b, win, wout):
    nch = win // 128
    return pl.pallas_call(
        _layer_body,
        grid=(N // BN,),
        in_specs=[
            pl.BlockSpec((NC, nch, BN, 128), lambda i: (0, 0, i, 0)),
            pl.BlockSpec((BN, win), lambda i: (i, 0)),
            pl.BlockSpec((BN, 1), lambda i: (i, 0)),
            pl.BlockSpec((win, wout), lambda i: (0, 0)),
            pl.BlockSpec((1, wout), lambda i: (0, 0)),
        ],
        out_specs=pl.BlockSpec((BN, wout), lambda i: (i, 0)),
        out_shape=jax.ShapeDtypeStruct((N, wout), jnp.float32),
    )(p, v, dinv, w, b)


def _pool_body(p_ref, v_ref, dinv_ref, bidx_ref, w3_ref, b3_ref,
               lw1_ref, lb1_ref, lw2_ref, lb2_ref, out_ref, ssum, scnt):
    i = pl.program_id(0)
    z = _combine(p_ref, v_ref, dinv_ref)
    onehot = (bidx_ref[...] == lax.broadcasted_iota(jnp.int32, (BN, G), 1))
    onehot = onehot.astype(jnp.float32)
    dn = (((0,), (0,)), ((), ()))
    part = lax.dot_general(onehot, z, dn, preferred_element_type=jnp.float32)
    cpart = lax.dot_general(
        onehot, jnp.ones((BN, 128), jnp.float32), dn,
        preferred_element_type=jnp.float32,
    )

    @pl.when(i == 0)
    def _():
        ssum[...] = jnp.zeros_like(ssum)
        scnt[...] = jnp.zeros_like(scnt)

    ssum[...] += part
    scnt[...] += cpart

    @pl.when(i == N // BN - 1)
    def _():
        pooled_z = ssum[...] / jnp.maximum(scnt[...][:, 0:1], 1.0)
        pooled = (
            jnp.dot(pooled_z, w3_ref[...], preferred_element_type=jnp.float32)
            + b3_ref[...]
        )
        f = jnp.maximum(
            jnp.dot(pooled, lw1_ref[...], preferred_element_type=jnp.float32)
            + lb1_ref[...],
            0.0,
        )
        out_ref[...] = (
            jnp.dot(f, lw2_ref[...], preferred_element_type=jnp.float32)
            + lb2_ref[...]
        )


def _tc_pool(p, v, dinv, bidx, w3, b3, lw1, lb1, lw2, lb2):
    return pl.pallas_call(
        _pool_body,
        grid=(N // BN,),
        in_specs=[
            pl.BlockSpec((NC, 4, BN, 128), lambda i: (0, 0, i, 0)),
            pl.BlockSpec((BN, H), lambda i: (i, 0)),
            pl.BlockSpec((BN, 1), lambda i: (i, 0)),
            pl.BlockSpec((BN, 1), lambda i: (i, 0)),
            pl.BlockSpec((H, H), lambda i: (0, 0)),
            pl.BlockSpec((1, H), lambda i: (0, 0)),
            pl.BlockSpec((H, G), lambda i: (0, 0)),
            pl.BlockSpec((1, G), lambda i: (0, 0)),
            pl.BlockSpec((G, 2), lambda i: (0, 0)),
            pl.BlockSpec((1, 2), lambda i: (0, 0)),
        ],
        out_specs=pl.BlockSpec((G, 2), lambda i: (0, 0)),
        out_shape=jax.ShapeDtypeStruct((G, 2), jnp.float32),
        scratch_shapes=[
            pltpu.VMEM((G, H), jnp.float32),
            pltpu.VMEM((G, 128), jnp.float32),
        ],
    )(p, v, dinv, bidx, w3, b3, lw1, lb1, lw2, lb2)


def kernel(x, edge_attr, edge_index, batch_index,
           W1, b1, W2, b2, W3, b3, LW1, Lb1, LW2, Lb2):
    del edge_attr
    src = edge_index[0]
    dst = edge_index[1]
    pad = E_PAD - E
    srcp = jnp.concatenate([src, jnp.zeros((pad,), jnp.int32)])
    dstp = jnp.concatenate([dst, jnp.full((pad,), N, jnp.int32)])
    dst2 = dstp.reshape(E_PAD // EB, EB)
    dst3d = dstp.reshape(NW, NDB, DB)

    ones_h = jnp.ones((DB, 128), jnp.float32)
    zeros_h = jnp.zeros((ZB, 128), jnp.float32)

    deg = _sc_degree(dst3d, ones_h, zeros_h)
    dinv, v1 = _tc_prep(deg, x)

    p1 = _sc_agg1(v1, srcp, dst2, zeros_h)
    v2 = _tc_layer(p1, v1, dinv, W1, b1.reshape(1, H), D, H)

    p2 = _sc_agg4(v2.reshape(N * 4, 128), srcp, dst2, zeros_h)
    v3 = _tc_layer(p2, v2, dinv, W2, b2.reshape(1, H), H, H)

    p3 = _sc_agg4(v3.reshape(N * 4, 128), srcp, dst2, zeros_h)
    out = _tc_pool(p3, v3, dinv,
                   batch_index.reshape(N, 1), W3, b3.reshape(1, H),
                   LW1, Lb1.reshape(1, G), LW2, Lb2.reshape(1, 2))
    return out


# speedup vs baseline: 1.4528x; 1.4528x over previous
import functools

import jax
import jax.numpy as jnp
from jax import lax
from jax.experimental import pallas as pl
from jax.experimental.pallas import tpu as pltpu
from jax.experimental.pallas import tpu_sc as plsc

N = 10000
E = 320000
D = 128
H = 512
G = 64

NC = 2
NS = 16
NW = NC * NS
EB = 128
NBUF = 2
STG = 2560
SB = STG // EB
NSTG0 = 7
NSTG1 = 1
EPW0 = NSTG0 * STG
EPW1 = NSTG1 * STG
DB = 128
NDB = 80
EPW = EPW0 + EPW1
E_PAD = NS * EPW
N_ACC = 10112
ACC_PER_TILE = N_ACC // NS
ZB = 8

_mesh = plsc.VectorSubcoreMesh(
    core_axis_name="c", subcore_axis_name="s", num_cores=NC, num_subcores=NS
)


def _wid(c, s):
    return s * NC + c


@functools.partial(
    pl.kernel,
    out_type=jax.ShapeDtypeStruct((NC, N_ACC, 128), jnp.float32),
    mesh=_mesh,
    scratch_types=[
        pltpu.VMEM((NDB, DB), jnp.int32),
        pltpu.VMEM((DB, 128), jnp.float32),
        pltpu.VMEM((ZB, 128), jnp.float32),
        pltpu.VMEM_SHARED((N_ACC, 128), jnp.float32),
    ],
)
def _sc_degree(dst3_hbm, ones_hbm, zeros_hbm, out_hbm, dstb, onesv, zb, acc):
    c = lax.axis_index("c")
    s = lax.axis_index("s")
    w = _wid(c, s)

    pltpu.sync_copy(dst3_hbm.at[w], dstb)
    pltpu.sync_copy(ones_hbm, onesv)
    pltpu.sync_copy(zeros_hbm, zb)
    for z in range(ACC_PER_TILE // ZB):
        pltpu.sync_copy(zb, acc.at[pl.ds(pl.multiple_of(s * ACC_PER_TILE + z * ZB, ZB), ZB)])
    plsc.subcore_barrier()

    def body(b, carry):
        pltpu.sync_copy(onesv, acc.at[dstb.at[b]], add=True)
        return carry

    lax.fori_loop(0, NDB, body, 0)
    plsc.subcore_barrier()
    pltpu.sync_copy(
        acc.at[pl.ds(pl.multiple_of(s * ACC_PER_TILE, 8), ACC_PER_TILE)],
        out_hbm.at[c, pl.ds(pl.multiple_of(s * ACC_PER_TILE, 8), ACC_PER_TILE)],
    )


def _make_sc_agg(nch):
    @functools.partial(
        pl.kernel,
        out_type=jax.ShapeDtypeStruct((NC, nch, N_ACC, 128), jnp.float32),
        mesh=_mesh,
        scratch_types=[
            pltpu.VMEM((STG,), jnp.int32),
            pltpu.VMEM((SB, EB), jnp.int32),
            pltpu.VMEM((NBUF, EB, 128), jnp.float32),
            pltpu.VMEM((ZB, 128), jnp.float32),
            pltpu.VMEM_SHARED((N_ACC, 128), jnp.float32),
            [pltpu.SemaphoreType.DMA] * NBUF,
            [pltpu.SemaphoreType.DMA] * NBUF,
        ],
    )
    def _sc_agg(vflat_hbm, src_hbm, dst2_hbm, zeros_hbm, out_hbm,
                gidx, dstb, rows, zb, acc, semg, sems):
        c = lax.axis_index("c")
        s = lax.axis_index("s")
        base = c * (NS * EPW0) + s * jnp.where(c == 0, EPW0, EPW1)
        nst = jnp.where(c == 0, NSTG0, NSTG1)

        pltpu.sync_copy(zeros_hbm, zb)

        def zero_acc():
            for z in range(ACC_PER_TILE // ZB):
                pltpu.async_copy(
                    zb, acc.at[pl.ds(pl.multiple_of(s * ACC_PER_TILE + z * ZB, ZB), ZB)], semg[0]
                )
            for z in range(ACC_PER_TILE // ZB):
                pltpu.make_async_copy(
                    zb, acc.at[pl.ds(pl.multiple_of(s * ACC_PER_TILE, ZB), ZB)], semg[0]
                ).wait()

        zero_acc()
        plsc.subcore_barrier()

        def gather_start(b, j):
            pltpu.async_copy(
                vflat_hbm.at[gidx.at[pl.ds(b * EB, EB)]], rows.at[j], semg[j]
            )

        def gather_wait(b, j):
            pltpu.make_async_copy(
                vflat_hbm.at[gidx.at[pl.ds(b * EB, EB)]], rows.at[j], semg[j]
            ).wait()

        def scat_start(b, j):
            pltpu.async_copy(
                rows.at[j], acc.at[dstb.at[b]], sems[j], add=True
            )

        def scat_wait(b, j):
            pltpu.make_async_copy(
                rows.at[j], acc.at[dstb.at[b]], sems[j]
            ).wait()

        def do_stage(k, stg):
                ebase = pl.multiple_of(base + stg * STG, STG)
                sidx = base // STG + stg
                pltpu.sync_copy(src_hbm.at[pl.ds(ebase, STG)], gidx)
                pltpu.sync_copy(dst2_hbm.at[sidx], dstb)
                if nch > 1:
                    def mkidx(jj, carry):
                        sl = pl.ds(jj * 16, 16)
                        gidx[sl] = gidx[sl] * nch + k
                        return carry

                    lax.fori_loop(0, STG // 16, mkidx, 0)

                gather_start(0, 0)

                def body(b2, carry):
                    for j in range(NBUF):
                        b = NBUF * b2 + j
                        jj = 1 - j
                        gather_wait(b, j)
                        scat_start(b, j)
                        if j == 1:
                            scat_wait(b - 1, jj)
                        else:

                            @pl.when(b2 > 0)
                            def _():
                                scat_wait(b - 1, jj)

                        @pl.when(b + 1 < SB)
                        def _():
                            gather_start(b + 1, jj)

                    return carry

                lax.fori_loop(0, SB // NBUF, body, 0)
                scat_wait(SB - 1, (SB - 1) % NBUF)

        for k in range(nch):
            for stg in range(max(NSTG0, NSTG1)):

                @pl.when(stg < nst)
                def _():
                    do_stage(k, stg)

            plsc.subcore_barrier()
            pltpu.sync_copy(
                acc.at[pl.ds(pl.multiple_of(s * ACC_PER_TILE, 8), ACC_PER_TILE)],
                out_hbm.at[c, k, pl.ds(pl.multiple_of(s * ACC_PER_TILE, 8), ACC_PER_TILE)],
            )
            if k + 1 < nch:
                zero_acc()
            plsc.subcore_barrier()

    return _sc_agg


_sc_agg1 = _make_sc_agg(1)
_sc_agg4 = _make_sc_agg(4)


BN = 1000


def _prep_body(deg_ref, x_ref, dinv_ref, v1_ref):
    cnt = deg_ref[0, :, 0:1] + deg_ref[1, :, 0:1]
    dinv = lax.rsqrt(1.0 + cnt)
    dinv_ref[...] = dinv
    v1_ref[...] = x_ref[...] * dinv


def _tc_prep(deg, x):
    return pl.pallas_call(
        _prep_body,
        grid=(N // BN,),
        in_specs=[
            pl.BlockSpec((NC, BN, 128), lambda i: (0, i, 0)),
            pl.BlockSpec((BN, D), lambda i: (i, 0)),
        ],
        out_specs=[
            pl.BlockSpec((BN, 1), lambda i: (i, 0)),
            pl.BlockSpec((BN, D), lambda i: (i, 0)),
        ],
        out_shape=[
            jax.ShapeDtypeStruct((N, 1), jnp.float32),
            jax.ShapeDtypeStruct((N, D), jnp.float32),
        ],
    )(deg, x)


def _combine(p_ref, v_ref, dinv_ref):
    nch = p_ref.shape[1]
    parts = [p_ref[0, k] + p_ref[1, k] for k in range(nch)]
    agg = parts[0] if nch == 1 else jnp.concatenate(parts, axis=1)
    return (agg + v_ref[...]) * dinv_ref[...]


def _layer_body(p_ref, v_ref, dinv_ref, w_ref, b_ref, out_ref):
    z = _combine(p_ref, v_ref, dinv_ref)
    h = jnp.dot(z, w_ref[...], preferred_element_type=jnp.float32) + b_ref[...]
    out_ref[...] = jnp.maximum(h, 0.0) * dinv_ref[...]


def _tc_layer(p, v, dinv, w, b, win, wout):
    nch = win // 128
    return pl.pallas_call(
        _layer_body,
        grid=(N // BN,),
        in_specs=[
            pl.BlockSpec((NC, nch, BN, 128), lambda i: (0, 0, i, 0)),
            pl.BlockSpec((BN, win), lambda i: (i, 0)),
            pl.BlockSpec((BN, 1), lambda i: (i, 0)),
            pl.BlockSpec((win, wout), lambda i: (0, 0)),
            pl.BlockSpec((1, wout), lambda i: (0, 0)),
        ],
        out_specs=pl.BlockSpec((BN, wout), lambda i: (i, 0)),
        out_shape=jax.ShapeDtypeStruct((N, wout), jnp.float32),
    )(p, v, dinv, w, b)


def _pool_body(p_ref, v_ref, dinv_ref, bidx_ref, w3_ref, b3_ref,
               lw1_ref, lb1_ref, lw2_ref, lb2_ref, out_ref, ssum, scnt):
    i = pl.program_id(0)
    z = _combine(p_ref, v_ref, dinv_ref)
    onehot = (bidx_ref[...] == lax.broadcasted_iota(jnp.int32, (BN, G), 1))
    onehot = onehot.astype(jnp.float32)
    dn = (((0,), (0,)), ((), ()))
    part = lax.dot_general(onehot, z, dn, preferred_element_type=jnp.float32)
    cpart = lax.dot_general(
        onehot, jnp.ones((BN, 128), jnp.float32), dn,
        preferred_element_type=jnp.float32,
    )

    @pl.when(i == 0)
    def _():
        ssum[...] = jnp.zeros_like(ssum)
        scnt[...] = jnp.zeros_like(scnt)

    ssum[...] += part
    scnt[...] += cpart

    @pl.when(i == N // BN - 1)
    def _():
        pooled_z = ssum[...] / jnp.maximum(scnt[...][:, 0:1], 1.0)
        pooled = (
            jnp.dot(pooled_z, w3_ref[...], preferred_element_type=jnp.float32)
            + b3_ref[...]
        )
        f = jnp.maximum(
            jnp.dot(pooled, lw1_ref[...], preferred_element_type=jnp.float32)
            + lb1_ref[...],
            0.0,
        )
        out_ref[...] = (
            jnp.dot(f, lw2_ref[...], preferred_element_type=jnp.float32)
            + lb2_ref[...]
        )


def _tc_pool(p, v, dinv, bidx, w3, b3, lw1, lb1, lw2, lb2):
    return pl.pallas_call(
        _pool_body,
        grid=(N // BN,),
        in_specs=[
            pl.BlockSpec((NC, 4, BN, 128), lambda i: (0, 0, i, 0)),
            pl.BlockSpec((BN, H), lambda i: (i, 0)),
            pl.BlockSpec((BN, 1), lambda i: (i, 0)),
            pl.BlockSpec((BN, 1), lambda i: (i, 0)),
            pl.BlockSpec((H, H), lambda i: (0, 0)),
            pl.BlockSpec((1, H), lambda i: (0, 0)),
            pl.BlockSpec((H, G), lambda i: (0, 0)),
            pl.BlockSpec((1, G), lambda i: (0, 0)),
            pl.BlockSpec((G, 2), lambda i: (0, 0)),
            pl.BlockSpec((1, 2), lambda i: (0, 0)),
        ],
        out_specs=pl.BlockSpec((G, 2), lambda i: (0, 0)),
        out_shape=jax.ShapeDtypeStruct((G, 2), jnp.float32),
        scratch_shapes=[
            pltpu.VMEM((G, H), jnp.float32),
            pltpu.VMEM((G, 128), jnp.float32),
        ],
    )(p, v, dinv, bidx, w3, b3, lw1, lb1, lw2, lb2)


def kernel(x, edge_attr, edge_index, batch_index,
           W1, b1, W2, b2, W3, b3, LW1, Lb1, LW2, Lb2):
    del edge_attr
    src = edge_index[0]
    dst = edge_index[1]
    pad = E_PAD - E
    srcp = jnp.concatenate([src, jnp.zeros((pad,), jnp.int32)])
    dstp = jnp.concatenate([dst, jnp.full((pad,), N, jnp.int32)])
    dst2 = dstp.reshape(E_PAD // STG, SB, EB)
    dst3d = dstp.reshape(NW, NDB, DB)

    ones_h = jnp.ones((DB, 128), jnp.float32)
    zeros_h = jnp.zeros((ZB, 128), jnp.float32)

    deg = _sc_degree(dst3d, ones_h, zeros_h)
    dinv, v1 = _tc_prep(deg, x)

    p1 = _sc_agg1(v1, srcp, dst2, zeros_h)
    v2 = _tc_layer(p1, v1, dinv, W1, b1.reshape(1, H), D, H)

    p2 = _sc_agg4(v2.reshape(N * 4, 128), srcp, dst2, zeros_h)
    v3 = _tc_layer(p2, v2, dinv, W2, b2.reshape(1, H), H, H)

    p3 = _sc_agg4(v3.reshape(N * 4, 128), srcp, dst2, zeros_h)
    out = _tc_pool(p3, v3, dinv,
                   batch_index.reshape(N, 1), W3, b3.reshape(1, H),
                   LW1, Lb1.reshape(1, G), LW2, Lb2.reshape(1, 2))
    return out
